# baseline (speedup 1.0000x reference)
import functools
import math

import jax
import jax.numpy as jnp
from jax import lax
from jax.experimental import pallas as pl
from jax.experimental.pallas import tpu as pltpu
from jax.experimental.pallas import tpu_sc as plsc

_ACT_SPARSITY = 0.65
_L = 16
_NW = 32
_UNROLL = 8

_TOP = 0x80000000


def _to_key(v):
    bits = lax.bitcast_convert_type(v, jnp.uint32)
    neg = (bits & jnp.uint32(_TOP)) != jnp.uint32(0)
    return jnp.where(neg, ~bits, bits | jnp.uint32(_TOP))


def _from_key(u):
    neg = (u & jnp.uint32(_TOP)) == jnp.uint32(0)
    bits = jnp.where(neg, ~u, u ^ jnp.uint32(_TOP))
    return lax.bitcast_convert_type(bits, jnp.float32)


def _suffix_scan(hist, kvec):
    lanes16 = jnp.zeros((_L,), jnp.int32)

    def body(i, carry):
        run, bstar, above, found = carry
        b = 255 - i
        h = hist[pl.ds(b * _L, _L)]
        newrun = run + h
        sel = jnp.logical_and(newrun >= kvec, jnp.logical_not(found))
        bvec = jnp.full((_L,), b, jnp.int32)
        bstar = jnp.where(sel, bvec, bstar)
        above = jnp.where(sel, run, above)
        found = jnp.logical_or(found, sel)
        return newrun, bstar, above, found

    init = (lanes16, lanes16, lanes16, jnp.zeros((_L,), jnp.bool_))
    _, bstar, above, _ = lax.fori_loop(0, 256, body, init)
    return bstar, above


def _sc_body(x_hbm, o_hbm, buf, hist, *, k, n_feat, groups_per_worker):
    nc = 2
    wid = lax.axis_index("s") * nc + lax.axis_index("c")
    lanes = lax.iota(jnp.int32, _L)
    ones = jnp.ones((_L,), jnp.int32)
    kvec0 = jnp.full((_L,), k, jnp.int32)
    steps = n_feat // _UNROLL

    def zero_hist():
        def zbody(i, _):
            hist[pl.ds(i * _L, _L)] = jnp.zeros((_L,), jnp.int32)
            return 0
        lax.fori_loop(0, 256, zbody, 0)

    def group_body(g, _):
        base = (wid * groups_per_worker + g) * _L

        pltpu.sync_copy(x_hbm.at[pl.ds(base * n_feat, _L * n_feat)], buf)

        zero_hist()

        def p1(i, _):
            for t in range(_UNROLL):
                j = i * _UNROLL + t
                idx = lanes * n_feat + j
                v = plsc.load_gather(buf, [idx])
                u = _to_key(v)
                plsc.store_scatter(
                    buf, [idx],
                    lax.bitcast_convert_type(u, jnp.float32))
                b = (u >> jnp.uint32(24)).astype(jnp.int32)
                plsc.addupdate_scatter(hist, [b * _L + lanes], ones)
            return 0
        lax.fori_loop(0, steps, p1, 0)

        bstar, above = _suffix_scan(hist, kvec0)
        kvec = kvec0 - above
        prefix = bstar.astype(jnp.uint32)

        for byte in (2, 1, 0):
            zero_hist()
            shift = jnp.uint32(8 * (byte + 1))
            bshift = jnp.uint32(8 * byte)

            def pn(i, _, shift=shift, bshift=bshift, prefix=prefix):
                for t in range(_UNROLL):
                    j = i * _UNROLL + t
                    idx = lanes * n_feat + j
                    v = plsc.load_gather(buf, [idx])
                    u = lax.bitcast_convert_type(v, jnp.uint32)
                    m = (u >> shift) == prefix
                    b = ((u >> bshift) & jnp.uint32(0xFF)).astype(jnp.int32)
                    plsc.addupdate_scatter(hist, [b * _L + lanes], ones,
                                           mask=m)
                return 0
            lax.fori_loop(0, steps, pn, 0)

            bstar, above = _suffix_scan(hist, kvec)
            kvec = kvec - above
            prefix = (prefix << jnp.uint32(8)) | bstar.astype(jnp.uint32)

        thresh = prefix

        def pm(i, _):
            for t in range(_UNROLL):
                j = i * _UNROLL + t
                idx = lanes * n_feat + j
                v = plsc.load_gather(buf, [idx])
                u = lax.bitcast_convert_type(v, jnp.uint32)
                out = jnp.where(u >= thresh, _from_key(u), jnp.float32(0.0))
                plsc.store_scatter(buf, [idx], out)
            return 0
        lax.fori_loop(0, steps, pm, 0)

        pltpu.sync_copy(buf, o_hbm.at[pl.ds(base * n_feat, _L * n_feat)])
        return 0

    lax.fori_loop(0, groups_per_worker, group_body, 0)


def kernel(inputs):
    n, f = inputs.shape
    k = math.floor((1.0 - _ACT_SPARSITY) * f)
    rows_per_worker = n // _NW
    groups_per_worker = rows_per_worker // _L
    mesh = plsc.VectorSubcoreMesh(core_axis_name="c", subcore_axis_name="s")
    body = functools.partial(
        _sc_body, k=k, n_feat=f, groups_per_worker=groups_per_worker)
    sc_fn = pl.kernel(
        body,
        mesh=mesh,
        out_type=jax.ShapeDtypeStruct((n * f,), inputs.dtype),
        scratch_types=[
            pltpu.VMEM((_L * f,), jnp.float32),
            pltpu.VMEM((256 * _L,), jnp.int32),
        ],
        compiler_params=pltpu.CompilerParams(needs_layout_passes=False),
    )
    return sc_fn(inputs.reshape(-1)).reshape(n, f)


# speedup vs baseline: 18.7044x; 18.7044x over previous
import functools
import math

import jax
import jax.numpy as jnp
from jax import lax
from jax.experimental import pallas as pl
from jax.experimental.pallas import tpu as pltpu
from jax.experimental.pallas import tpu_sc as plsc

_ACT_SPARSITY = 0.65
_L = 16
_NW = 32
_UNROLL = 8

_TOP = 0x80000000


def _to_key(v):
    bits = lax.bitcast_convert_type(v, jnp.uint32)
    neg = (bits & jnp.uint32(_TOP)) != jnp.uint32(0)
    return jnp.where(neg, ~bits, bits | jnp.uint32(_TOP))


def _from_key(u):
    neg = (u & jnp.uint32(_TOP)) == jnp.uint32(0)
    bits = jnp.where(neg, ~u, u ^ jnp.uint32(_TOP))
    return lax.bitcast_convert_type(bits, jnp.float32)


def _suffix_scan(hist, kvec):
    zeros = jnp.zeros((_L,), jnp.int32)

    def body(i, carry):
        run, bstar, above, found = carry
        for t in range(_UNROLL):
            b = 255 - (i * _UNROLL + t)
            h = hist[pl.ds(b * _L, _L)]
            newrun = run + h
            sel = jnp.logical_and(newrun >= kvec, jnp.logical_not(found))
            bvec = jnp.full((_L,), b, jnp.int32)
            bstar = jnp.where(sel, bvec, bstar)
            above = jnp.where(sel, run, above)
            found = jnp.logical_or(found, sel)
            run = newrun
        return run, bstar, above, found

    init = (zeros, zeros, zeros, jnp.zeros((_L,), jnp.bool_))
    _, bstar, above, _ = lax.fori_loop(0, 256 // _UNROLL, body, init)
    return bstar, above


def _sc_body(x_hbm, o_hbm, buf, kbuf, hist, *, k, n_feat, groups_per_worker):
    nc = 2
    wid = lax.axis_index("s") * nc + lax.axis_index("c")
    lanes = lax.iota(jnp.int32, _L)
    base_idx = lanes * n_feat
    ones = jnp.ones((_L,), jnp.int32)
    kvec0 = jnp.full((_L,), k, jnp.int32)

    def zero_hist():
        @functools.partial(
            plsc.parallel_loop, 0, 256, unroll=_UNROLL)
        def _(i):
            hist[pl.ds(i * _L, _L)] = jnp.zeros((_L,), jnp.int32)

    def group_body(g, _):
        base = (wid * groups_per_worker + g) * _L

        pltpu.sync_copy(x_hbm.at[pl.ds(base * n_feat, _L * n_feat)], buf)

        zero_hist()

        @functools.partial(plsc.parallel_loop, 0, n_feat, unroll=_UNROLL)
        def _(j):
            idx = base_idx + j
            v = plsc.load_gather(buf, [idx])
            u = _to_key(v)
            plsc.store_scatter(kbuf, [idx],
                               lax.bitcast_convert_type(u, jnp.int32))
            b = (u >> jnp.uint32(24)).astype(jnp.int32)
            plsc.addupdate_scatter(hist, [b * _L + lanes], ones)

        bstar, above = _suffix_scan(hist, kvec0)
        kvec = kvec0 - above
        prefix = bstar.astype(jnp.uint32)

        for byte in (2, 1, 0):
            zero_hist()
            shift = jnp.uint32(8 * (byte + 1))
            bshift = jnp.uint32(8 * byte)

            @functools.partial(plsc.parallel_loop, 0, n_feat, unroll=_UNROLL)
            def _(j, shift=shift, bshift=bshift, prefix=prefix):
                idx = base_idx + j
                u = lax.bitcast_convert_type(
                    plsc.load_gather(kbuf, [idx]), jnp.uint32)
                m = (u >> shift) == prefix
                b = ((u >> bshift) & jnp.uint32(0xFF)).astype(jnp.int32)
                plsc.addupdate_scatter(hist, [b * _L + lanes], ones, mask=m)

            bstar, above = _suffix_scan(hist, kvec)
            kvec = kvec - above
            prefix = (prefix << jnp.uint32(8)) | bstar.astype(jnp.uint32)

        thresh = prefix

        @functools.partial(plsc.parallel_loop, 0, n_feat, unroll=_UNROLL)
        def _(j):
            idx = base_idx + j
            u = lax.bitcast_convert_type(
                plsc.load_gather(kbuf, [idx]), jnp.uint32)
            out = jnp.where(u >= thresh, _from_key(u), jnp.float32(0.0))
            plsc.store_scatter(buf, [idx], out)

        pltpu.sync_copy(buf, o_hbm.at[pl.ds(base * n_feat, _L * n_feat)])
        return 0

    lax.fori_loop(0, groups_per_worker, group_body, 0)


def kernel(inputs):
    n, f = inputs.shape
    k = math.floor((1.0 - _ACT_SPARSITY) * f)
    rows_per_worker = n // _NW
    groups_per_worker = rows_per_worker // _L
    mesh = plsc.VectorSubcoreMesh(core_axis_name="c", subcore_axis_name="s")
    body = functools.partial(
        _sc_body, k=k, n_feat=f, groups_per_worker=groups_per_worker)
    sc_fn = pl.kernel(
        body,
        mesh=mesh,
        out_type=jax.ShapeDtypeStruct((n * f,), inputs.dtype),
        scratch_types=[
            pltpu.VMEM((_L * f,), jnp.float32),
            pltpu.VMEM((_L * f,), jnp.int32),
            pltpu.VMEM((256 * _L,), jnp.int32),
        ],
        compiler_params=pltpu.CompilerParams(needs_layout_passes=False),
    )
    return sc_fn(inputs.reshape(-1)).reshape(n, f)
